# baseline (speedup 1.0000x reference)
import functools

import jax
import jax.numpy as jnp
from jax import lax
from jax.experimental import pallas as pl
from jax.experimental.pallas import tpu as pltpu
from jax.experimental.pallas import tpu_sc as plsc

NUM_FIELDS = 26
DIM = 32
B = 16384
NC, NS = 2, 16
NW = NC * NS
CHUNK = 128
ROWS_PER_W = B // NW
CPW = ROWS_PER_W // CHUNK


@jax.jit
def _sc_embed(idxs, tables):
    mesh = plsc.VectorSubcoreMesh(core_axis_name="c", subcore_axis_name="s")

    @functools.partial(
        pl.kernel,
        out_type=jax.ShapeDtypeStruct((B, NUM_FIELDS * DIM), jnp.float32),
        mesh=mesh,
        scratch_types=[
            pltpu.VMEM((CPW, CHUNK), jnp.int32),
            pltpu.VMEM((CHUNK, DIM), jnp.float32),
            pltpu.SemaphoreType.DMA,
        ],
        compiler_params=pltpu.CompilerParams(use_tc_tiling_on_sc=False),
    )
    def k(*refs):
        idx_refs = refs[:NUM_FIELDS]
        tab_refs = refs[NUM_FIELDS:2 * NUM_FIELDS]
        out = refs[2 * NUM_FIELDS]
        idx_v, rows_v, sem = refs[2 * NUM_FIELDS + 1:]
        wid = lax.axis_index("s") * NC + lax.axis_index("c")
        row0 = wid * CPW
        b0 = wid * ROWS_PER_W
        for f in range(NUM_FIELDS):
            pltpu.sync_copy(idx_refs[f].at[pl.ds(row0, CPW)], idx_v)
            for c in range(CPW):
                pltpu.async_copy(tab_refs[f].at[idx_v.at[c]], rows_v, sem).wait()
                pltpu.sync_copy(
                    rows_v,
                    out.at[pl.ds(b0 + c * CHUNK, CHUNK), pl.ds(f * DIM, DIM)],
                )

    return k(*idxs, *tables)


def kernel(f00, f01, f02, f03, f04, f05, f06, f07, f08, f09, f10, f11, f12,
           f13, f14, f15, f16, f17, f18, f19, f20, f21, f22, f23, f24, f25,
           W00, W01, W02, W03, W04, W05, W06, W07, W08, W09, W10, W11, W12,
           W13, W14, W15, W16, W17, W18, W19, W20, W21, W22, W23, W24, W25):
    idxs = [f00, f01, f02, f03, f04, f05, f06, f07, f08, f09, f10, f11, f12,
            f13, f14, f15, f16, f17, f18, f19, f20, f21, f22, f23, f24, f25]
    tables = [W00, W01, W02, W03, W04, W05, W06, W07, W08, W09, W10, W11, W12,
              W13, W14, W15, W16, W17, W18, W19, W20, W21, W22, W23, W24, W25]
    idxs = [idx.reshape(B // CHUNK, CHUNK) for idx in idxs]
    out = _sc_embed(idxs, tables)
    return out.reshape(B, NUM_FIELDS, DIM)


# speedup vs baseline: 1.0647x; 1.0647x over previous
import functools

import jax
import jax.numpy as jnp
from jax import lax
from jax.experimental import pallas as pl
from jax.experimental.pallas import tpu as pltpu
from jax.experimental.pallas import tpu_sc as plsc

NUM_FIELDS = 26
DIM = 32
B = 16384
NC, NS = 2, 16
NW = NC * NS
CHUNK = 128
ROWS_PER_W = B // NW
CPW = ROWS_PER_W // CHUNK


NBUF = 6
LAG = 3
NTASK = NUM_FIELDS * CPW


@jax.jit
def _sc_embed(idxs, tables):
    mesh = plsc.VectorSubcoreMesh(core_axis_name="c", subcore_axis_name="s")

    @functools.partial(
        pl.kernel,
        out_type=jax.ShapeDtypeStruct((B, NUM_FIELDS * DIM), jnp.float32),
        mesh=mesh,
        scratch_types=[
            pltpu.VMEM((NUM_FIELDS, CPW, CHUNK), jnp.int32),
            pltpu.VMEM((NBUF, CHUNK, DIM), jnp.float32),
            pltpu.SemaphoreType.DMA((NBUF,)),
            pltpu.SemaphoreType.DMA((NBUF,)),
        ],
        compiler_params=pltpu.CompilerParams(use_tc_tiling_on_sc=False),
    )
    def k(*refs):
        idx_refs = refs[:NUM_FIELDS]
        tab_refs = refs[NUM_FIELDS:2 * NUM_FIELDS]
        out = refs[2 * NUM_FIELDS]
        idx_v, bufs, gsem, ssem = refs[2 * NUM_FIELDS + 1:]
        wid = lax.axis_index("s") * NC + lax.axis_index("c")
        row0 = wid * CPW
        b0 = wid * ROWS_PER_W

        for f in range(NUM_FIELDS):
            pltpu.sync_copy(idx_refs[f].at[pl.ds(row0, CPW)], idx_v.at[f])

        ghandles = [None] * NBUF
        shandles = [None] * NBUF

        def start_gather(t):
            f, c = divmod(t, CPW)
            s = t % NBUF
            ghandles[s] = pltpu.async_copy(
                tab_refs[f].at[idx_v.at[f, c]], bufs.at[s], gsem.at[s])

        def retire_gather_start_store(t):
            f, c = divmod(t, CPW)
            s = t % NBUF
            ghandles[s].wait()
            shandles[s] = pltpu.async_copy(
                bufs.at[s],
                out.at[pl.ds(b0 + c * CHUNK, CHUNK), pl.ds(f * DIM, DIM)],
                ssem.at[s])

        for t in range(NTASK):
            s = t % NBUF
            if t >= NBUF:
                shandles[s].wait()
            start_gather(t)
            if t >= LAG:
                retire_gather_start_store(t - LAG)
        for u in range(NTASK - LAG, NTASK):
            retire_gather_start_store(u)
        for u in range(NTASK - NBUF, NTASK):
            shandles[u % NBUF].wait()

    return k(*idxs, *tables)


def kernel(f00, f01, f02, f03, f04, f05, f06, f07, f08, f09, f10, f11, f12,
           f13, f14, f15, f16, f17, f18, f19, f20, f21, f22, f23, f24, f25,
           W00, W01, W02, W03, W04, W05, W06, W07, W08, W09, W10, W11, W12,
           W13, W14, W15, W16, W17, W18, W19, W20, W21, W22, W23, W24, W25):
    idxs = [f00, f01, f02, f03, f04, f05, f06, f07, f08, f09, f10, f11, f12,
            f13, f14, f15, f16, f17, f18, f19, f20, f21, f22, f23, f24, f25]
    tables = [W00, W01, W02, W03, W04, W05, W06, W07, W08, W09, W10, W11, W12,
              W13, W14, W15, W16, W17, W18, W19, W20, W21, W22, W23, W24, W25]
    idxs = [idx.reshape(B // CHUNK, CHUNK) for idx in idxs]
    out = _sc_embed(idxs, tables)
    return out.reshape(B, NUM_FIELDS, DIM)
